# baseline (speedup 1.0000x reference)
import jax
import jax.numpy as jnp
import numpy as np
from jax.experimental import pallas as pl
from jax.experimental.pallas import tpu as pltpu
from jax.experimental.pallas import tpu_sc as plsc

_S = 1024
_N = 50
_NMAX = 64
_D = 64
_EPS = 0.1
_E_FAC = np.float32(1.0 / np.sqrt(1.0 + _EPS ** 2))
_C_NOISE = np.float32(_EPS / np.sqrt(_D))
_P = 2 * _NMAX + 1
_T = 2 * _N + 1

_NW = 32
_C = 4
_SCH = _S // _C
_SPW = _SCH // _NW
_RC = _SPW * (_N + 1)
_RL = _SPW * _N
_RPAD = 56
_B = 16


def _pieces(r):
    out = []
    off = 0
    while off < r:
        sz = min(128, r - off)
        out.append((off, sz))
        off += sz
    return out


def _sc_gather_chunk(k, cls_wide, lab_wide, idx_cls, idx_lab):
    base_cls = k * _SCH * (_N + 1)
    base_lab = k * _SCH * _N
    mesh = plsc.VectorSubcoreMesh(core_axis_name="c", subcore_axis_name="s")

    @pl.kernel(
        out_type=(
            jax.ShapeDtypeStruct((_SCH, _RPAD, 2 * _D), jnp.float32),
            jax.ShapeDtypeStruct((_SCH, _RPAD, 2 * _D), jnp.float32),
        ),
        mesh=mesh,
        scratch_types=[
            pltpu.VMEM((_RC + 8,), jnp.int32),
            pltpu.VMEM((_RC + 8,), jnp.int32),
            pltpu.VMEM((_RC + 8,), jnp.int32),
            pltpu.VMEM((_RC + 8,), jnp.int32),
            pltpu.VMEM((_RC + 8, 2 * _D), jnp.float32),
            pltpu.VMEM((_RL + 8, 2 * _D), jnp.float32),
            pltpu.SemaphoreType.DMA,
            pltpu.SemaphoreType.DMA,
        ],
    )
    def k_fn(cls_hbm, lab_hbm, ic_hbm, il_hbm, oc_hbm, ol_hbm,
             idxc_v, halfc_v, idxl_v, halfl_v, stc_v, stl_v, gsem, osem):
        wid = jax.lax.axis_index("s") * 2 + jax.lax.axis_index("c")
        s0 = wid * _SPW

        def load_shift(i_hbm, base, r, idx_v, half_v):
            pltpu.sync_copy(i_hbm.at[pl.ds(base + wid * r, r)],
                            idx_v.at[pl.ds(0, r)])
            for c in range(-(-r // 16)):
                sl = pl.ds(c * 16, 16)
                half_v[sl] = jax.lax.shift_right_logical(idx_v[sl], 1)

        def fire_gathers(tab, half_v, st_v, r):
            return [
                pltpu.async_copy(tab.at[half_v.at[pl.ds(off, sz)]],
                                 st_v.at[pl.ds(off, sz)], gsem)
                for off, sz in _pieces(r)
            ]

        def copy_out(st_v, o_hbm, rows):
            return [
                pltpu.async_copy(st_v.at[pl.ds(t * rows, _RPAD)],
                                 o_hbm.at[s0 + t], osem)
                for t in range(_SPW)
            ]

        load_shift(ic_hbm, base_cls, _RC, idxc_v, halfc_v)
        hc = fire_gathers(cls_hbm, halfc_v, stc_v, _RC)
        load_shift(il_hbm, base_lab, _RL, idxl_v, halfl_v)
        hl = fire_gathers(lab_hbm, halfl_v, stl_v, _RL)
        for h in hc:
            h.wait()
        oc = copy_out(stc_v, oc_hbm, _N + 1)
        for h in hl:
            h.wait()
        ol = copy_out(stl_v, ol_hbm, _N)
        for h in oc + ol:
            h.wait()

    return k_fn(cls_wide, lab_wide, idx_cls, idx_lab)


def _assemble_body(shift_ref, ex_ref, lb_ref, cls_ref, lab_ref,
                   nc_ref, nq_ref, out_ref):
    clsw = cls_ref[...][:, :_N + 1, :]
    labw = lab_ref[...][:, :_N, :]
    exp = (ex_ref[...] & 1)[:, :, None]
    lbp = (lb_ref[...] & 1)[:, :, None]
    cls = jnp.where(exp == 1, clsw[:, :, _D:], clsw[:, :, :_D])
    lab = jnp.where(lbp == 1, labw[:, :, _D:], labw[:, :, :_D])
    nc = nc_ref[...]
    nq = nq_ref[...].reshape(_B, 1, _D)
    ctx = _E_FAC * (cls[:, :_N, :] + _C_NOISE * nc)
    q = _E_FAC * (cls[:, _N:, :] + _C_NOISE * nq)
    pair = jnp.stack([ctx, lab], axis=2).reshape(_B, 2 * _N, _D)
    feat = jnp.concatenate([pair, q], axis=1)
    sh = shift_ref[...].reshape(_B, 1, 1)
    row = jax.lax.broadcasted_iota(jnp.int32, (_B, _T, _P), 1)
    col = jax.lax.broadcasted_iota(jnp.int32, (_B, _T, _P), 2)
    pos = (col == sh + row).astype(jnp.float32)
    out_ref[...] = jnp.concatenate([pos, feat], axis=2)


def _assemble_chunk(k, shifts2, example2, label2, cls3, lab3,
                    noise_ctx, noise_q):
    base = k * (_SCH // _B)
    grid = (_SCH // _B,)

    def full(i):
        return (base + i, 0, 0)

    def full2(i):
        return (base + i, 0)

    def local(i):
        return (i, 0, 0)

    in_specs = [
        pl.BlockSpec((_B, 1), full2),
        pl.BlockSpec((_B, _N + 1), full2),
        pl.BlockSpec((_B, _N), full2),
        pl.BlockSpec((_B, _RPAD, 2 * _D), local),
        pl.BlockSpec((_B, _RPAD, 2 * _D), local),
        pl.BlockSpec((_B, _N, _D), full),
        pl.BlockSpec((_B, _D), full2),
    ]
    args = [shifts2, example2, label2, cls3, lab3, noise_ctx, noise_q]
    return pl.pallas_call(
        _assemble_body,
        grid=grid,
        in_specs=in_specs,
        out_specs=pl.BlockSpec((_B, _T, _P + _D), local),
        out_shape=jax.ShapeDtypeStruct((_SCH, _T, _P + _D), jnp.float32),
    )(*args)


def kernel(example, label, noise_ctx, noise_q, shifts, mus_label, mus_class):
    example = example.astype(jnp.int32)
    label = label.astype(jnp.int32)
    idx_cls = example.reshape(-1)
    idx_lab = label[:, :_N].reshape(-1)
    cls_wide = mus_class.reshape(-1, 2 * _D)
    lab_wide = mus_label.reshape(-1, 2 * _D)
    shifts2 = shifts.astype(jnp.int32).reshape(_S, 1)

    chunks = []
    for k in range(_C):
        cls3, lab3 = _sc_gather_chunk(k, cls_wide, lab_wide,
                                      idx_cls, idx_lab)
        chunks.append(_assemble_chunk(k, shifts2, example, label[:, :_N],
                                      cls3, lab3, noise_ctx, noise_q))
    return jnp.concatenate(chunks, axis=0)


# speedup vs baseline: 2.1297x; 2.1297x over previous
import dataclasses

import jax
import jax.numpy as jnp
import numpy as np
from jax.experimental import pallas as pl
from jax.experimental.pallas import tpu as pltpu
from jax.experimental.pallas import tpu_sc as plsc

_S = 1024
_N = 50
_NMAX = 64
_D = 64
_K = 100000
_EPS = 0.1
_E_FAC = np.float32(1.0 / np.sqrt(1.0 + _EPS ** 2))
_C_NOISE = np.float32(_EPS / np.sqrt(_D))
_P = 2 * _NMAX + 1
_T = 2 * _N + 1

_NW = 32
_RPAD = 56
_NIP = _RPAD * _S
_CH = 8
_NCH = _RPAD // _CH
_DPW = _D // (_NW // 2)
_B = 128


def _sc_gather_t(cls_t, lab_t, idxc, idxl):
    mesh = plsc.VectorSubcoreMesh(core_axis_name="c", subcore_axis_name="s")
    cp = pltpu.CompilerParams()
    if "needs_layout_passes" in pltpu.CompilerParams.__dataclass_fields__:
        cp = dataclasses.replace(cp, needs_layout_passes=False)

    @pl.kernel(
        out_type=(
            jax.ShapeDtypeStruct((_D, _RPAD, _S), jnp.float32),
            jax.ShapeDtypeStruct((_D, _RPAD, _S), jnp.float32),
        ),
        mesh=mesh,
        compiler_params=cp,
        scratch_types=[
            pltpu.VMEM((_K,), jnp.float32),
            pltpu.VMEM((_CH * _S,), jnp.int32),
            pltpu.VMEM((_CH, _S), jnp.float32),
            pltpu.VMEM((_CH, _S), jnp.float32),
            pltpu.SemaphoreType.DMA,
            pltpu.SemaphoreType.DMA,
        ],
    )
    def k_fn(cls_hbm, lab_hbm, ic_hbm, il_hbm, oc_hbm, ol_hbm,
             row_v, idx_v, outa_v, outb_v, rsem, osem):
        wid = jax.lax.axis_index("s") * 2 + jax.lax.axis_index("c")
        outs = (outa_v, outb_v)

        def task(tab_hbm, i_hbm, o_hbm, d):
            pltpu.async_copy(tab_hbm.at[d], row_v, rsem).wait()
            handles = [None, None]
            for c in range(_NCH):
                b = c % 2
                pltpu.sync_copy(i_hbm.at[pl.ds(c * _CH * _S, _CH * _S)],
                                idx_v)
                if handles[b] is not None:
                    handles[b].wait()
                o_v = outs[b]

                @pl.loop(0, _CH * _S // 16, step=8)
                def _(j0):
                    for u in range(8):
                        j = j0 + u
                        idx16 = idx_v[pl.ds(j * 16, 16)]
                        vals = plsc.load_gather(row_v, [idx16])
                        r = j // (_S // 16)
                        col = (j % (_S // 16)) * 16
                        o_v[r, pl.ds(col, 16)] = vals

                handles[b] = pltpu.async_copy(
                    o_v, o_hbm.at[d, pl.ds(c * _CH, _CH)], osem)
            for h in handles:
                if h is not None:
                    h.wait()

        half = wid % (_NW // 2)

        @pl.when(wid < _NW // 2)
        def _():
            for m in range(_DPW):
                task(cls_hbm, ic_hbm, oc_hbm, half * _DPW + m)

        @pl.when(wid >= _NW // 2)
        def _():
            for m in range(_DPW):
                task(lab_hbm, il_hbm, ol_hbm, half * _DPW + m)

    return k_fn(cls_t, lab_t, idxc, idxl)


def _assemble_body_t(shift_ref, gc_ref, gl_ref, nc_ref, nq_ref, out_ref):
    gc = gc_ref[...][:, :_N + 1, :]
    gl = gl_ref[...][:, :_N, :]
    nc = jnp.swapaxes(nc_ref[...], 0, 1)
    nq = nq_ref[...]
    ctx = _E_FAC * (gc[:, :_N, :] + _C_NOISE * nc)
    q = _E_FAC * (gc[:, _N, :] + _C_NOISE * nq)
    pair = jnp.stack([ctx, gl], axis=2).reshape(_D, 2 * _N, _B)
    feat = jnp.concatenate([pair, q[:, None, :]], axis=1)
    sh = shift_ref[...].reshape(1, 1, _B)
    j_io = jax.lax.broadcasted_iota(jnp.int32, (_P, _T, _B), 0)
    t_io = jax.lax.broadcasted_iota(jnp.int32, (_P, _T, _B), 1)
    pos = (j_io == t_io + sh).astype(jnp.float32)
    out_ref[pl.ds(0, _P)] = pos
    out_ref[pl.ds(_P, _D)] = feat


def _assemble_t(shifts, gc_t, gl_t, nc_t, nq_t):
    grid = (_S // _B,)
    return pl.pallas_call(
        _assemble_body_t,
        grid=grid,
        in_specs=[
            pl.BlockSpec((_B,), lambda i: (i,)),
            pl.BlockSpec((_D, _RPAD, _B), lambda i: (0, 0, i)),
            pl.BlockSpec((_D, _RPAD, _B), lambda i: (0, 0, i)),
            pl.BlockSpec((_N, _D, _B), lambda i: (0, 0, i)),
            pl.BlockSpec((_D, _B), lambda i: (0, i)),
        ],
        out_specs=pl.BlockSpec((_P + _D, _T, _B), lambda i: (0, 0, i)),
        out_shape=jax.ShapeDtypeStruct((_P + _D, _T, _S), jnp.float32),
    )(shifts, gc_t, gl_t, nc_t, nq_t)


def kernel(example, label, noise_ctx, noise_q, shifts, mus_label, mus_class):
    example = example.astype(jnp.int32)
    label = label.astype(jnp.int32)
    idxc = jnp.pad(example.T.reshape(-1), (0, _NIP - _S * (_N + 1)))
    idxl = jnp.pad(label.T[:_N].reshape(-1), (0, _NIP - _S * _N))
    cls_t = mus_class.T
    lab_t = mus_label.T
    nc_t = noise_ctx.transpose(1, 2, 0)
    nq_t = noise_q.T

    gc_t, gl_t = _sc_gather_t(cls_t, lab_t, idxc, idxl)
    out_t = _assemble_t(shifts.astype(jnp.int32), gc_t, gl_t, nc_t, nq_t)
    return out_t.transpose(2, 1, 0)


# speedup vs baseline: 2.3497x; 1.1033x over previous
import dataclasses

import jax
import jax.numpy as jnp
import numpy as np
from jax.experimental import pallas as pl
from jax.experimental.pallas import tpu as pltpu
from jax.experimental.pallas import tpu_sc as plsc

_S = 1024
_N = 50
_NMAX = 64
_D = 64
_K = 100000
_EPS = 0.1
_E_FAC = np.float32(1.0 / np.sqrt(1.0 + _EPS ** 2))
_C_NOISE = np.float32(_EPS / np.sqrt(_D))
_P = 2 * _NMAX + 1
_T = 2 * _N + 1

_NW = 32
_RPAD = 56
_NIP = _RPAD * _S
_CH = 8
_ICH = 4
_NIC = _RPAD // _ICH
_DPW = _D // (_NW // 2)
_B = 128


def _sc_gather_t(cls_t, lab_t, idxc, idxl):
    mesh = plsc.VectorSubcoreMesh(core_axis_name="c", subcore_axis_name="s")
    cp = pltpu.CompilerParams()
    if "needs_layout_passes" in pltpu.CompilerParams.__dataclass_fields__:
        cp = dataclasses.replace(cp, needs_layout_passes=False)

    @pl.kernel(
        out_type=(
            jax.ShapeDtypeStruct((_D, _RPAD, _S), jnp.float32),
            jax.ShapeDtypeStruct((_D, _RPAD, _S), jnp.float32),
        ),
        mesh=mesh,
        compiler_params=cp,
        scratch_types=[
            pltpu.VMEM((_K,), jnp.float32),
            pltpu.VMEM((_ICH * _S,), jnp.int32),
            pltpu.VMEM((_ICH * _S,), jnp.int32),
            pltpu.VMEM((_CH, _S), jnp.float32),
            pltpu.VMEM((_CH, _S), jnp.float32),
            pltpu.SemaphoreType.DMA,
            pltpu.SemaphoreType.DMA,
            pltpu.SemaphoreType.DMA,
        ],
    )
    def k_fn(cls_hbm, lab_hbm, ic_hbm, il_hbm, oc_hbm, ol_hbm,
             row_v, idxa_v, idxb_v, outa_v, outb_v, rsem, isem, osem):
        wid = jax.lax.axis_index("s") * 2 + jax.lax.axis_index("c")
        idxs = (idxa_v, idxb_v)
        outs = (outa_v, outb_v)

        def task(tab_hbm, i_hbm, o_hbm, d):
            rh = pltpu.async_copy(tab_hbm.at[d], row_v, rsem)
            ih = [None, None]
            oh = [None, None]
            ih[0] = pltpu.async_copy(i_hbm.at[pl.ds(0, _ICH * _S)],
                                     idxa_v, isem)
            rh.wait()
            for c in range(_NIC):
                b = c % 2
                ih[b].wait()
                if c + 1 < _NIC:
                    ih[1 - b] = pltpu.async_copy(
                        i_hbm.at[pl.ds((c + 1) * _ICH * _S, _ICH * _S)],
                        idxs[1 - b], isem)
                half = c % 2
                ob = (c // 2) % 2
                o_v = outs[ob]
                if half == 0 and oh[ob] is not None:
                    oh[ob].wait()
                i_v = idxs[b]

                @pl.loop(0, _ICH * _S // 16, step=8)
                def _(j0):
                    for u in range(8):
                        j = j0 + u
                        idx16 = i_v[pl.ds(j * 16, 16)]
                        vals = plsc.load_gather(row_v, [idx16])
                        jj = j + half * (_ICH * _S // 16)
                        r = jj // (_S // 16)
                        col = (jj % (_S // 16)) * 16
                        o_v[r, pl.ds(col, 16)] = vals

                if half == 1:
                    oh[ob] = pltpu.async_copy(
                        o_v, o_hbm.at[d, pl.ds((c // 2) * _CH, _CH)], osem)
            for h in oh:
                if h is not None:
                    h.wait()

        half = wid % (_NW // 2)

        @pl.when(wid < _NW // 2)
        def _():
            for m in range(_DPW):
                task(cls_hbm, ic_hbm, oc_hbm, half * _DPW + m)

        @pl.when(wid >= _NW // 2)
        def _():
            for m in range(_DPW):
                task(lab_hbm, il_hbm, ol_hbm, half * _DPW + m)

    return k_fn(cls_t, lab_t, idxc, idxl)


def _assemble_body_t(shift_ref, gc_ref, gl_ref, nc_ref, nq_ref, out_ref):
    gc = gc_ref[...][:, :_N + 1, :]
    gl = gl_ref[...][:, :_N, :]
    nc = jnp.swapaxes(nc_ref[...], 0, 1)
    nq = nq_ref[...]
    ctx = _E_FAC * (gc[:, :_N, :] + _C_NOISE * nc)
    q = _E_FAC * (gc[:, _N, :] + _C_NOISE * nq)
    pair = jnp.stack([ctx, gl], axis=2).reshape(_D, 2 * _N, _B)
    feat = jnp.concatenate([pair, q[:, None, :]], axis=1)
    sh = shift_ref[...].reshape(1, 1, _B)
    j_io = jax.lax.broadcasted_iota(jnp.int32, (_P, _T, _B), 0)
    t_io = jax.lax.broadcasted_iota(jnp.int32, (_P, _T, _B), 1)
    pos = (j_io == t_io + sh).astype(jnp.float32)
    out_ref[pl.ds(0, _P)] = pos
    out_ref[pl.ds(_P, _D)] = feat


def _assemble_t(shifts, gc_t, gl_t, nc_t, nq_t):
    grid = (_S // _B,)
    return pl.pallas_call(
        _assemble_body_t,
        grid=grid,
        in_specs=[
            pl.BlockSpec((_B,), lambda i: (i,)),
            pl.BlockSpec((_D, _RPAD, _B), lambda i: (0, 0, i)),
            pl.BlockSpec((_D, _RPAD, _B), lambda i: (0, 0, i)),
            pl.BlockSpec((_N, _D, _B), lambda i: (0, 0, i)),
            pl.BlockSpec((_D, _B), lambda i: (0, i)),
        ],
        out_specs=pl.BlockSpec((_P + _D, _T, _B), lambda i: (0, 0, i)),
        out_shape=jax.ShapeDtypeStruct((_P + _D, _T, _S), jnp.float32),
    )(shifts, gc_t, gl_t, nc_t, nq_t)


def kernel(example, label, noise_ctx, noise_q, shifts, mus_label, mus_class):
    example = example.astype(jnp.int32)
    label = label.astype(jnp.int32)
    idxc = jnp.pad(example.T.reshape(-1), (0, _NIP - _S * (_N + 1)))
    idxl = jnp.pad(label.T[:_N].reshape(-1), (0, _NIP - _S * _N))
    cls_t = mus_class.T
    lab_t = mus_label.T
    nc_t = noise_ctx.transpose(1, 2, 0)
    nq_t = noise_q.T

    gc_t, gl_t = _sc_gather_t(cls_t, lab_t, idxc, idxl)
    out_t = _assemble_t(shifts.astype(jnp.int32), gc_t, gl_t, nc_t, nq_t)
    return out_t.transpose(2, 1, 0)
